# baseline (speedup 1.0000x reference)
import functools

import jax
import jax.numpy as jnp
from jax import lax
from jax.experimental import pallas as pl
from jax.experimental.pallas import tpu as pltpu
from jax.experimental.pallas import tpu_sc as plsc

_F = 26
_V = 20
_E = 64
_A = 20.0
_NV = _F * _V
_NVP = 528


def _tc_prep(xt_ref, primt_ref, dfltt_ref, cnt_ref, blendt_ref, fidxt_ref):
    c = cnt_ref[...].astype(jnp.float32)
    w = (c / (c + _A))[None, :]
    blendt_ref[...] = w * primt_ref[...] + (1.0 - w) * dfltt_ref[...]
    fofs = lax.broadcasted_iota(jnp.int32, xt_ref.shape, 0) * _V
    fidxt_ref[...] = xt_ref[...] + fofs


def kernel(X, emb_table, counts):
    B = X.shape[0]
    NBT = B // 128

    emb3 = emb_table.reshape(_F, _V + 1, _E)
    primt = jnp.transpose(emb3[:, 1:, :], (2, 0, 1)).reshape(_E, _NV)
    dfltt = jnp.repeat(jnp.transpose(emb3[:, 0, :], (1, 0)), _V, axis=1)
    primt = jnp.pad(primt, ((0, 0), (0, _NVP - _NV)))
    dfltt = jnp.pad(dfltt, ((0, 0), (0, _NVP - _NV)))
    cntp = jnp.pad(counts.reshape(_NV), (0, _NVP - _NV))
    XT = jnp.transpose(X, (1, 0))

    blendt, fidxt = pl.pallas_call(
        _tc_prep,
        out_shape=(
            jax.ShapeDtypeStruct((_E, _NVP), jnp.float32),
            jax.ShapeDtypeStruct((_F, B), jnp.int32),
        ),
    )(XT, primt, dfltt, cntp)

    info = plsc.get_sparse_core_info()
    NC, NS = info.num_cores, info.num_subcores
    NW = NC * NS
    NCHUNK = _F * NBT
    CPW = NCHUNK // NW
    fidx2 = fidxt.reshape(NCHUNK, 128)

    mesh = plsc.VectorSubcoreMesh(core_axis_name="c", subcore_axis_name="s")

    @functools.partial(
        pl.kernel,
        out_type=jax.ShapeDtypeStruct((_F, 8, NBT, 8, 128), jnp.float32),
        mesh=mesh,
        compiler_params=pltpu.CompilerParams(
            use_tc_tiling_on_sc=False, needs_layout_passes=False
        ),
        scratch_types=[
            pltpu.VMEM((_E, _NVP), jnp.float32),
            pltpu.VMEM((CPW, 128), jnp.int32),
            pltpu.VMEM((8, 8, 128), jnp.float32),
            pltpu.VMEM((8, 8, 128), jnp.float32),
            pltpu.VMEM((8, 8, 128), jnp.float32),
            pltpu.VMEM((8, 8, 128), jnp.float32),
            pltpu.SemaphoreType.DMA,
            pltpu.SemaphoreType.DMA,
            pltpu.SemaphoreType.DMA,
            pltpu.SemaphoreType.DMA,
        ],
    )
    def sc_fill(
        fidx_hbm, blendt_hbm, out_hbm, tbl_v, idx_v,
        obuf0, obuf1, obuf2, obuf3, sem0, sem1, sem2, sem3,
    ):
        wid = lax.axis_index("s") * NC + lax.axis_index("c")
        pltpu.sync_copy(blendt_hbm, tbl_v)
        pltpu.sync_copy(fidx_hbm.at[pl.ds(wid * CPW, CPW)], idx_v)

        def out_slice(t):
            return out_hbm.at[t // NBT, :, t % NBT]

        def chunk(j, obuf, sem, wait_prev):
            t = wid * CPW + j
            if wait_prev:
                @pl.when(j >= 4)
                def _():
                    pltpu.make_async_copy(obuf, out_slice(t - 4), sem).wait()

            for c in range(8):
                idx16 = idx_v[j, pl.ds(c * 16, 16)]
                for eg in range(0, _E, 8):
                    vals = [
                        plsc.load_gather(
                            tbl_v, [jnp.full((16,), e, jnp.int32), idx16]
                        )
                        for e in range(eg, eg + 8)
                    ]
                    for k, v in enumerate(vals):
                        e = eg + k
                        obuf[e // 8, e % 8, pl.ds(c * 16, 16)] = v
            pltpu.async_copy(obuf, out_slice(t), sem)

        chunk(0, obuf0, sem0, wait_prev=False)
        chunk(1, obuf1, sem1, wait_prev=False)

        def body(i, carry):
            j0 = 2 * i + 2

            @pl.when(i % 2 == 0)
            def _():
                chunk(j0, obuf2, sem2, wait_prev=True)
                chunk(j0 + 1, obuf3, sem3, wait_prev=True)

            @pl.when(i % 2 == 1)
            def _():
                chunk(j0, obuf0, sem0, wait_prev=True)
                chunk(j0 + 1, obuf1, sem1, wait_prev=True)

            return carry

        lax.fori_loop(0, (CPW - 2) // 2, body, 0)
        base = wid * CPW
        pltpu.make_async_copy(obuf2, out_slice(base + CPW - 4), sem2).wait()
        pltpu.make_async_copy(obuf3, out_slice(base + CPW - 3), sem3).wait()
        pltpu.make_async_copy(obuf0, out_slice(base + CPW - 2), sem0).wait()
        pltpu.make_async_copy(obuf1, out_slice(base + CPW - 1), sem1).wait()

    q = sc_fill(fidx2, blendt)
    return q.transpose((2, 4, 0, 1, 3)).reshape(B, _F, _E)


# speedup vs baseline: 1.2709x; 1.2709x over previous
import functools

import jax
import jax.numpy as jnp
from jax import lax
from jax.experimental import pallas as pl
from jax.experimental.pallas import tpu as pltpu
from jax.experimental.pallas import tpu_sc as plsc

_F = 26
_V = 20
_VP = 32
_E = 64
_A = 20.0
_NT = _F * _VP


def _dg(a16, i16):
    return lax.gather(
        a16,
        i16[:, None],
        lax.GatherDimensionNumbers(
            offset_dims=(), collapsed_slice_dims=(0,), start_index_map=(0,)
        ),
        (1,),
        mode=lax.GatherScatterMode.PROMISE_IN_BOUNDS,
    )


def _tc_prep(primt_ref, dfltt_ref, cnt_ref, blendt_ref):
    c = cnt_ref[...].astype(jnp.float32)
    w = (c / (c + _A))[None, :]
    blendt_ref[...] = w * primt_ref[...] + (1.0 - w) * dfltt_ref[...]


def kernel(X, emb_table, counts):
    B = X.shape[0]
    NBT = B // 128

    emb3 = emb_table.reshape(_F, _V + 1, _E)
    primt = jnp.transpose(emb3[:, 1:, :], (2, 0, 1))
    dfltt = jnp.broadcast_to(
        jnp.transpose(emb3[:, 0, :], (1, 0))[:, :, None], (_E, _F, _V)
    )
    primt = jnp.pad(primt, ((0, 0), (0, 0), (0, _VP - _V))).reshape(_E, _NT)
    dfltt = jnp.pad(dfltt, ((0, 0), (0, 0), (0, _VP - _V))).reshape(_E, _NT)
    cntp = jnp.pad(counts, ((0, 0), (0, _VP - _V))).reshape(_NT)

    blendt = pl.pallas_call(
        _tc_prep,
        out_shape=jax.ShapeDtypeStruct((_E, _NT), jnp.float32),
    )(primt, dfltt, cntp)

    info = plsc.get_sparse_core_info()
    NC, NS = info.num_cores, info.num_subcores
    NW = NC * NS
    NCHUNK = _F * NBT
    CPW = NCHUNK // NW
    fidx2 = jnp.transpose(X, (1, 0)).reshape(NCHUNK, 128)

    mesh = plsc.VectorSubcoreMesh(core_axis_name="c", subcore_axis_name="s")

    @functools.partial(
        pl.kernel,
        out_type=jax.ShapeDtypeStruct((_F, 8, NBT, 8, 128), jnp.float32),
        mesh=mesh,
        compiler_params=pltpu.CompilerParams(
            use_tc_tiling_on_sc=False, needs_layout_passes=False
        ),
        scratch_types=[
            pltpu.VMEM((_E, _NT), jnp.float32),
            pltpu.VMEM((CPW, 128), jnp.int32),
            pltpu.VMEM((8, 8, 128), jnp.float32),
            pltpu.VMEM((8, 8, 128), jnp.float32),
            pltpu.SemaphoreType.DMA,
            pltpu.SemaphoreType.DMA,
        ],
    )
    def sc_fill(fidx_hbm, blendt_hbm, out_hbm, tbl_v, idx_v, obuf0, obuf1, sem0, sem1):
        wid = lax.axis_index("s") * NC + lax.axis_index("c")
        pltpu.sync_copy(blendt_hbm, tbl_v)
        pltpu.sync_copy(fidx_hbm.at[pl.ds(wid * CPW, CPW)], idx_v)

        def out_slice(t):
            return out_hbm.at[t // NBT, :, t % NBT]

        def chunk(j, obuf, sem):
            t = wid * CPW + j
            f = t // NBT
            fbase = f * _VP

            @pl.when(j >= 2)
            def _():
                pltpu.make_async_copy(obuf, out_slice(t - 2), sem).wait()

            xs, ms = [], []
            for c in range(8):
                x16 = idx_v[j, pl.ds(c * 16, 16)]
                xs.append(x16 & 15)
                ms.append(x16 < 16)
            for e in range(_E):
                lo = tbl_v[e, pl.ds(fbase, 16)]
                hi = tbl_v[e, pl.ds(fbase + 16, 16)]
                for c in range(8):
                    obuf[e // 8, e % 8, pl.ds(c * 16, 16)] = jnp.where(
                        ms[c], _dg(lo, xs[c]), _dg(hi, xs[c])
                    )
            pltpu.async_copy(obuf, out_slice(t), sem)

        def body(i, carry):
            chunk(2 * i, obuf0, sem0)
            chunk(2 * i + 1, obuf1, sem1)
            return carry

        lax.fori_loop(0, CPW // 2, body, 0)
        base = wid * CPW
        pltpu.make_async_copy(obuf0, out_slice(base + CPW - 2), sem0).wait()
        pltpu.make_async_copy(obuf1, out_slice(base + CPW - 1), sem1).wait()

    q = sc_fill(fidx2, blendt)
    return q.transpose((2, 4, 0, 1, 3)).reshape(B, _F, _E)


# speedup vs baseline: 1.9048x; 1.4988x over previous
import functools

import jax
import jax.numpy as jnp
from jax import lax
from jax.experimental import pallas as pl
from jax.experimental.pallas import tpu as pltpu
from jax.experimental.pallas import tpu_sc as plsc

_F = 26
_V = 20
_VP = 32
_E = 64
_A = 20.0
_NT = _F * _VP


def _dg(a16, i16):
    return lax.gather(
        a16,
        i16[:, None],
        lax.GatherDimensionNumbers(
            offset_dims=(), collapsed_slice_dims=(0,), start_index_map=(0,)
        ),
        (1,),
        mode=lax.GatherScatterMode.PROMISE_IN_BOUNDS,
    )


def _tc_prep(primt_ref, dfltt_ref, cnt_ref, blendt_ref):
    c = cnt_ref[...].astype(jnp.float32)
    w = (c / (c + _A))[None, :]
    blendt_ref[...] = w * primt_ref[...] + (1.0 - w) * dfltt_ref[...]


def kernel(X, emb_table, counts):
    B = X.shape[0]
    NBT = B // 128

    emb3 = emb_table.reshape(_F, _V + 1, _E)
    primt = jnp.transpose(emb3[:, 1:, :], (2, 0, 1))
    dfltt = jnp.broadcast_to(
        jnp.transpose(emb3[:, 0, :], (1, 0))[:, :, None], (_E, _F, _V)
    )
    primt = jnp.pad(primt, ((0, 0), (0, 0), (0, _VP - _V))).reshape(_E, _NT)
    dfltt = jnp.pad(dfltt, ((0, 0), (0, 0), (0, _VP - _V))).reshape(_E, _NT)
    cntp = jnp.pad(counts, ((0, 0), (0, _VP - _V))).reshape(_NT)

    blendt = pl.pallas_call(
        _tc_prep,
        out_shape=jax.ShapeDtypeStruct((_E, _NT), jnp.float32),
    )(primt, dfltt, cntp)

    info = plsc.get_sparse_core_info()
    NC, NS = info.num_cores, info.num_subcores
    NW = NC * NS
    NCHUNK = _F * NBT
    CPW = NCHUNK // NW
    fidx2 = jnp.transpose(X, (1, 0)).reshape(NCHUNK, 128)

    mesh = plsc.VectorSubcoreMesh(core_axis_name="c", subcore_axis_name="s")

    @functools.partial(
        pl.kernel,
        out_type=jax.ShapeDtypeStruct((_F, 8, NBT, 8, 128), jnp.float32),
        mesh=mesh,
        compiler_params=pltpu.CompilerParams(
            use_tc_tiling_on_sc=False, needs_layout_passes=False
        ),
        scratch_types=[
            pltpu.VMEM((_E, _NT), jnp.float32),
            pltpu.VMEM((CPW, 128), jnp.int32),
            pltpu.VMEM((8, 8, 128), jnp.float32),
            pltpu.VMEM((8, 8, 128), jnp.float32),
            pltpu.SemaphoreType.DMA,
            pltpu.SemaphoreType.DMA,
        ],
    )
    def sc_fill(fidx_hbm, blendt_hbm, out_hbm, tbl_v, idx_v, obuf0, obuf1, sem0, sem1):
        wid = lax.axis_index("s") * NC + lax.axis_index("c")
        pltpu.sync_copy(blendt_hbm, tbl_v)
        pltpu.sync_copy(fidx_hbm.at[pl.ds(wid * CPW, CPW)], idx_v)

        def out_slice(t):
            return out_hbm.at[t // NBT, :, t % NBT]

        def chunk(j, obuf, sem):
            t = wid * CPW + j
            f = t // NBT
            fbase = f * _VP

            @pl.when(j >= 2)
            def _():
                pltpu.make_async_copy(obuf, out_slice(t - 2), sem).wait()

            xs, ms = [], []
            for c in range(8):
                x16 = idx_v[j, pl.ds(c * 16, 16)]
                xs.append(x16 & 15)
                ms.append(x16 < 16)
            lo = tbl_v[0, pl.ds(fbase, 16)]
            hi = tbl_v[0, pl.ds(fbase + 16, 16)]
            for e in range(_E):
                if e + 1 < _E:
                    lo_n = tbl_v[e + 1, pl.ds(fbase, 16)]
                    hi_n = tbl_v[e + 1, pl.ds(fbase + 16, 16)]
                for c in range(8):
                    obuf[e // 8, e % 8, pl.ds(c * 16, 16)] = jnp.where(
                        ms[c], _dg(lo, xs[c]), _dg(hi, xs[c])
                    )
                if e + 1 < _E:
                    lo, hi = lo_n, hi_n
            pltpu.async_copy(obuf, out_slice(t), sem)

        def body(i, carry):
            chunk(2 * i, obuf0, sem0)
            chunk(2 * i + 1, obuf1, sem1)
            return carry

        lax.fori_loop(0, CPW // 2, body, 0)
        base = wid * CPW
        pltpu.make_async_copy(obuf0, out_slice(base + CPW - 2), sem0).wait()
        pltpu.make_async_copy(obuf1, out_slice(base + CPW - 1), sem1).wait()

    q = sc_fill(fidx2, blendt)
    return q.transpose((2, 4, 0, 1, 3)).reshape(B, _F, _E)
